# baseline (speedup 1.0000x reference)
import jax
import jax.numpy as jnp
from jax import lax
from jax.experimental import pallas as pl
from jax.experimental.pallas import tpu as pltpu
from jax.experimental.pallas import tpu_sc as plsc

N = 10000
E = 160000
F = 256
H = 256
C = 40
G = 16

NC = 2
NS = 16
CH = 128

NP = 10240
ROWS_PER_TILE = NP // NS
NCHUNK = 80
EP = NS * NCHUNK * CH
BN = 1024
GRID = NP // BN

_mesh = plsc.VectorSubcoreMesh(
    core_axis_name="c", subcore_axis_name="s", num_cores=NC, num_subcores=NS
)


def _count_body(dst_hbm, ones_hbm, zeros_hbm, cnt_hbm, dst_v, ones_v, shared):
    c = lax.axis_index("c")
    s = lax.axis_index("s")
    r0 = s * ROWS_PER_TILE

    @pl.when(c == 0)
    def _():
        pltpu.sync_copy(zeros_hbm, shared.at[pl.ds(r0, ROWS_PER_TILE)])
        pltpu.sync_copy(dst_hbm.at[s], dst_v)
        pltpu.sync_copy(ones_hbm, ones_v)

    plsc.subcore_barrier()

    @pl.when(c == 0)
    def _():
        def body(j, carry):
            pltpu.sync_copy(ones_v, shared.at[dst_v.at[j]], add=True)
            return carry
        lax.fori_loop(0, NCHUNK, body, 0)

    plsc.subcore_barrier()

    @pl.when(c == 0)
    def _():
        pltpu.sync_copy(shared.at[pl.ds(r0, ROWS_PER_TILE)],
                        cnt_hbm.at[pl.ds(r0, ROWS_PER_TILE)])


_count_kernel = pl.kernel(
    _count_body,
    mesh=_mesh,
    out_type=jax.ShapeDtypeStruct((NP, 128), jnp.float32),
    scratch_types=[
        pltpu.VMEM((NCHUNK, CH), jnp.int32),
        pltpu.VMEM((CH, 128), jnp.float32),
        pltpu.VMEM_SHARED((NP, 128), jnp.float32),
    ],
)


NBUF = 2
NPH = 2
CPP = NCHUNK // NPH


def _segsum_body(ya_hbm, src_hbm, dst_hbm, zeros_hbm, agg_hbm,
                 src_v, dst_v, buf, gsems, ssems, shared):
    c = lax.axis_index("c")
    s = lax.axis_index("s")
    r0 = s * ROWS_PER_TILE

    pltpu.sync_copy(zeros_hbm, shared.at[pl.ds(r0, ROWS_PER_TILE)])
    plsc.subcore_barrier()

    def gather_start(j, b):
        pltpu.async_copy(ya_hbm.at[src_v.at[j]], buf.at[b], gsems.at[b])

    def gather_wait(j, b):
        pltpu.make_async_copy(ya_hbm.at[src_v.at[j]], buf.at[b],
                              gsems.at[b]).wait()

    def scatter_start(j, b):
        pltpu.async_copy(buf.at[b], shared.at[dst_v.at[j]], ssems.at[b],
                         add=True)

    def scatter_wait(j, b):
        pltpu.make_async_copy(buf.at[b], shared.at[dst_v.at[j]],
                              ssems.at[b]).wait()

    for ph in range(NPH):
        pltpu.sync_copy(src_hbm.at[c, s, pl.ds(ph * CPP, CPP)], src_v)
        pltpu.sync_copy(dst_hbm.at[s, pl.ds(ph * CPP, CPP)], dst_v)

        gather_start(0, 0)

        def body(jj, carry):
            j0 = NBUF * jj
            for b in range(NBUF):
                j = j0 + b
                ob = (b + 1) % NBUF
                gather_wait(j, b)

                @pl.when(j >= 1)
                def _():
                    scatter_wait(j - 1, ob)

                @pl.when(j + 1 < CPP)
                def _():
                    gather_start(j + 1, ob)

                scatter_start(j, b)
            return carry

        lax.fori_loop(0, CPP // NBUF, body, 0)
        scatter_wait(CPP - 1, (CPP - 1) % NBUF)

    plsc.subcore_barrier()

    pltpu.sync_copy(shared.at[pl.ds(r0, ROWS_PER_TILE)],
                    agg_hbm.at[c, pl.ds(r0, ROWS_PER_TILE)])


_segsum_kernel = pl.kernel(
    _segsum_body,
    mesh=_mesh,
    out_type=jax.ShapeDtypeStruct((NC, NP, 128), jnp.float32),
    scratch_types=[
        pltpu.VMEM((CPP, CH), jnp.int32),
        pltpu.VMEM((CPP, CH), jnp.int32),
        pltpu.VMEM((NBUF, CH, 128), jnp.float32),
        pltpu.SemaphoreType.DMA((NBUF,)),
        pltpu.SemaphoreType.DMA((NBUF,)),
        pltpu.VMEM_SHARED((NP, 128), jnp.float32),
    ],
)


def _lin_body(x_ref, wl_ref, wr_ref, bl_ref, ya_ref, z_ref):
    xb = x_ref[...]
    y = jax.lax.dot_general(xb, wl_ref[...], (((1,), (1,)), ((), ())),
                            preferred_element_type=jnp.float32)
    ya_ref[0] = y[:, :128]
    ya_ref[1] = y[:, 128:]
    z_ref[...] = jax.lax.dot_general(xb, wr_ref[...], (((1,), (1,)), ((), ())),
                                     preferred_element_type=jnp.float32) + bl_ref[...]


def _mid_body(agg_ref, cnt_ref, z_ref, wl_ref, wr_ref, bl_ref,
              ya_ref, zn_ref):
    d = 1.0 / jnp.maximum(cnt_ref[:, :1], 1.0)
    h = jnp.concatenate([agg_ref[0], agg_ref[1]], axis=1) * d + z_ref[...]
    h = jnp.maximum(h, 0.0)
    y = jax.lax.dot_general(h, wl_ref[...], (((1,), (1,)), ((), ())),
                            preferred_element_type=jnp.float32)
    ya_ref[0] = y[:, :128]
    ya_ref[1] = y[:, 128:]
    zn_ref[...] = jax.lax.dot_general(h, wr_ref[...], (((1,), (1,)), ((), ())),
                                      preferred_element_type=jnp.float32) + bl_ref[...]


def _pool_body(agg_ref, cnt_ref, z_ref, p_ref, wfc_ref, bfc_ref, out_ref,
               psum, pcnt):
    i = pl.program_id(0)

    @pl.when(i == 0)
    def _():
        psum[...] = jnp.zeros_like(psum)
        pcnt[...] = jnp.zeros_like(pcnt)

    d = 1.0 / jnp.maximum(cnt_ref[:, :1], 1.0)
    h = jnp.concatenate([agg_ref[0], agg_ref[1]], axis=1) * d + z_ref[...]
    p = p_ref[...]
    psum[...] += jax.lax.dot_general(p, h, (((1,), (0,)), ((), ())),
                                     preferred_element_type=jnp.float32)
    pcnt[...] += jnp.sum(p, axis=1, keepdims=True)

    @pl.when(i == GRID - 1)
    def _():
        pooled = psum[...] / jnp.maximum(pcnt[:, :1], 1.0)
        out_ref[...] = jax.lax.dot_general(
            pooled, wfc_ref[...], (((1,), (1,)), ((), ())),
            preferred_element_type=jnp.float32) + bfc_ref[...]


def _lin_call(xp, Wl, Wr, bl):
    return pl.pallas_call(
        _lin_body,
        grid=(GRID,),
        in_specs=[
            pl.BlockSpec((BN, F), lambda i: (i, 0)),
            pl.BlockSpec((H, F), lambda i: (0, 0)),
            pl.BlockSpec((H, F), lambda i: (0, 0)),
            pl.BlockSpec((1, H), lambda i: (0, 0)),
        ],
        out_specs=[
            pl.BlockSpec((NC, BN, 128), lambda i: (0, i, 0)),
            pl.BlockSpec((BN, H), lambda i: (i, 0)),
        ],
        out_shape=[
            jax.ShapeDtypeStruct((NC, NP, 128), jnp.float32),
            jax.ShapeDtypeStruct((NP, H), jnp.float32),
        ],
    )(xp, Wl, Wr, bl.reshape(1, H))


def _mid_call(agg, cnt, z, Wl, Wr, bl):
    return pl.pallas_call(
        _mid_body,
        grid=(GRID,),
        in_specs=[
            pl.BlockSpec((NC, BN, 128), lambda i: (0, i, 0)),
            pl.BlockSpec((BN, 128), lambda i: (i, 0)),
            pl.BlockSpec((BN, H), lambda i: (i, 0)),
            pl.BlockSpec((H, H), lambda i: (0, 0)),
            pl.BlockSpec((H, H), lambda i: (0, 0)),
            pl.BlockSpec((1, H), lambda i: (0, 0)),
        ],
        out_specs=[
            pl.BlockSpec((NC, BN, 128), lambda i: (0, i, 0)),
            pl.BlockSpec((BN, H), lambda i: (i, 0)),
        ],
        out_shape=[
            jax.ShapeDtypeStruct((NC, NP, 128), jnp.float32),
            jax.ShapeDtypeStruct((NP, H), jnp.float32),
        ],
    )(agg, cnt, z, Wl, Wr, bl.reshape(1, H))


def _pool_call(agg, cnt, z, P01, Wfc, bfc):
    return pl.pallas_call(
        _pool_body,
        grid=(GRID,),
        in_specs=[
            pl.BlockSpec((NC, BN, 128), lambda i: (0, i, 0)),
            pl.BlockSpec((BN, 128), lambda i: (i, 0)),
            pl.BlockSpec((BN, H), lambda i: (i, 0)),
            pl.BlockSpec((G, BN), lambda i: (0, i)),
            pl.BlockSpec((C, H), lambda i: (0, 0)),
            pl.BlockSpec((1, C), lambda i: (0, 0)),
        ],
        out_specs=pl.BlockSpec((G, C), lambda i: (0, 0)),
        out_shape=jax.ShapeDtypeStruct((G, C), jnp.float32),
        scratch_shapes=[
            pltpu.VMEM((G, H), jnp.float32),
            pltpu.VMEM((G, 128), jnp.float32),
        ],
    )(agg, cnt, z, P01, Wfc, bfc.reshape(1, C))


def kernel(x, edge_index, batch, Wl1, bl1, Wr1, Wl2, bl2, Wr2, Wl3, bl3, Wr3,
           Wfc, bfc):
    xp = jnp.pad(x, ((0, NP - N), (0, 0)))
    src_flat = jnp.pad(edge_index[0], (0, EP - E))
    src = jnp.stack([src_flat, src_flat + NP]).reshape(NC, NS, NCHUNK, CH)
    dst = jnp.pad(edge_index[1], (0, EP - E), constant_values=N).reshape(
        NS, NCHUNK, CH)
    batch_p = jnp.pad(batch, (0, NP - N), constant_values=G)
    P01 = (batch_p[None, :] == jnp.arange(G, dtype=batch.dtype)[:, None]
           ).astype(jnp.float32)
    ones128 = jnp.ones((CH, 128), jnp.float32)
    zeros128 = jnp.zeros((ROWS_PER_TILE, 128), jnp.float32)

    cnt = _count_kernel(dst, ones128, zeros128)

    ya, z = _lin_call(xp, Wl1, Wr1, bl1)
    agg = _segsum_kernel(ya.reshape(NC * NP, 128), src, dst, zeros128)
    ya, z = _mid_call(agg, cnt, z, Wl2, Wr2, bl2)
    agg = _segsum_kernel(ya.reshape(NC * NP, 128), src, dst, zeros128)
    ya, z = _mid_call(agg, cnt, z, Wl3, Wr3, bl3)
    agg = _segsum_kernel(ya.reshape(NC * NP, 128), src, dst, zeros128)
    return _pool_call(agg, cnt, z, P01, Wfc, bfc)


# speedup vs baseline: 1.0379x; 1.0379x over previous
import jax
import jax.numpy as jnp
from jax import lax
from jax.experimental import pallas as pl
from jax.experimental.pallas import tpu as pltpu
from jax.experimental.pallas import tpu_sc as plsc

N = 10000
E = 160000
F = 256
H = 256
C = 40
G = 16

NC = 2
NS = 16
CH = 64

NP = 10240
ROWS_PER_TILE = NP // NS
NCHUNK = 160
EP = NS * NCHUNK * CH
BN = 1024
GRID = NP // BN

_mesh = plsc.VectorSubcoreMesh(
    core_axis_name="c", subcore_axis_name="s", num_cores=NC, num_subcores=NS
)


def _count_body(dst_hbm, ones_hbm, zeros_hbm, cnt_hbm, dst_v, ones_v, shared):
    c = lax.axis_index("c")
    s = lax.axis_index("s")
    r0 = s * ROWS_PER_TILE

    @pl.when(c == 0)
    def _():
        pltpu.sync_copy(zeros_hbm, shared.at[pl.ds(r0, ROWS_PER_TILE)])
        pltpu.sync_copy(dst_hbm.at[s], dst_v)
        pltpu.sync_copy(ones_hbm, ones_v)

    plsc.subcore_barrier()

    @pl.when(c == 0)
    def _():
        def body(j, carry):
            pltpu.sync_copy(ones_v, shared.at[dst_v.at[j]], add=True)
            return carry
        lax.fori_loop(0, NCHUNK, body, 0)

    plsc.subcore_barrier()

    @pl.when(c == 0)
    def _():
        pltpu.sync_copy(shared.at[pl.ds(r0, ROWS_PER_TILE)],
                        cnt_hbm.at[pl.ds(r0, ROWS_PER_TILE)])


_count_kernel = pl.kernel(
    _count_body,
    mesh=_mesh,
    out_type=jax.ShapeDtypeStruct((NP, 128), jnp.float32),
    scratch_types=[
        pltpu.VMEM((NCHUNK, CH), jnp.int32),
        pltpu.VMEM((CH, 128), jnp.float32),
        pltpu.VMEM_SHARED((NP, 128), jnp.float32),
    ],
)


NBUF = 4
NPH = 4
CPP = NCHUNK // NPH


def _segsum_body(ya_hbm, src_hbm, dst_hbm, zeros_hbm, agg_hbm,
                 src_v, dst_v, buf, gsems, ssems, shared):
    c = lax.axis_index("c")
    s = lax.axis_index("s")
    r0 = s * ROWS_PER_TILE

    pltpu.sync_copy(zeros_hbm, shared.at[pl.ds(r0, ROWS_PER_TILE)])
    plsc.subcore_barrier()

    def gather_start(j, b):
        pltpu.async_copy(ya_hbm.at[src_v.at[j]], buf.at[b], gsems.at[b])

    def gather_wait(j, b):
        pltpu.make_async_copy(ya_hbm.at[src_v.at[j]], buf.at[b],
                              gsems.at[b]).wait()

    def scatter_start(j, b):
        pltpu.async_copy(buf.at[b], shared.at[dst_v.at[j]], ssems.at[b],
                         add=True)

    def scatter_wait(j, b):
        pltpu.make_async_copy(buf.at[b], shared.at[dst_v.at[j]],
                              ssems.at[b]).wait()

    for ph in range(NPH):
        pltpu.sync_copy(src_hbm.at[c, s, pl.ds(ph * CPP, CPP)], src_v)
        pltpu.sync_copy(dst_hbm.at[s, pl.ds(ph * CPP, CPP)], dst_v)

        gather_start(0, 0)
        gather_start(1, 1)

        def body(jj, carry):
            j0 = NBUF * jj
            for b in range(NBUF):
                j = j0 + b
                gather_wait(j, b)
                scatter_start(j, b)

                @pl.when(j >= 2)
                def _():
                    scatter_wait(j - 2, (b + 2) % NBUF)

                @pl.when(j + 2 < CPP)
                def _():
                    gather_start(j + 2, (b + 2) % NBUF)

            return carry

        lax.fori_loop(0, CPP // NBUF, body, 0)
        scatter_wait(CPP - 2, (CPP - 2) % NBUF)
        scatter_wait(CPP - 1, (CPP - 1) % NBUF)

    plsc.subcore_barrier()

    pltpu.sync_copy(shared.at[pl.ds(r0, ROWS_PER_TILE)],
                    agg_hbm.at[c, pl.ds(r0, ROWS_PER_TILE)])


_segsum_kernel = pl.kernel(
    _segsum_body,
    mesh=_mesh,
    out_type=jax.ShapeDtypeStruct((NC, NP, 128), jnp.float32),
    scratch_types=[
        pltpu.VMEM((CPP, CH), jnp.int32),
        pltpu.VMEM((CPP, CH), jnp.int32),
        pltpu.VMEM((NBUF, CH, 128), jnp.float32),
        pltpu.SemaphoreType.DMA((NBUF,)),
        pltpu.SemaphoreType.DMA((NBUF,)),
        pltpu.VMEM_SHARED((NP, 128), jnp.float32),
    ],
)


def _lin_body(x_ref, wl_ref, wr_ref, bl_ref, ya_ref, z_ref):
    xb = x_ref[...]
    y = jax.lax.dot_general(xb, wl_ref[...], (((1,), (1,)), ((), ())),
                            preferred_element_type=jnp.float32)
    ya_ref[0] = y[:, :128]
    ya_ref[1] = y[:, 128:]
    z_ref[...] = jax.lax.dot_general(xb, wr_ref[...], (((1,), (1,)), ((), ())),
                                     preferred_element_type=jnp.float32) + bl_ref[...]


def _mid_body(agg_ref, cnt_ref, z_ref, wl_ref, wr_ref, bl_ref,
              ya_ref, zn_ref):
    d = 1.0 / jnp.maximum(cnt_ref[:, :1], 1.0)
    h = jnp.concatenate([agg_ref[0], agg_ref[1]], axis=1) * d + z_ref[...]
    h = jnp.maximum(h, 0.0)
    y = jax.lax.dot_general(h, wl_ref[...], (((1,), (1,)), ((), ())),
                            preferred_element_type=jnp.float32)
    ya_ref[0] = y[:, :128]
    ya_ref[1] = y[:, 128:]
    zn_ref[...] = jax.lax.dot_general(h, wr_ref[...], (((1,), (1,)), ((), ())),
                                      preferred_element_type=jnp.float32) + bl_ref[...]


def _pool_body(agg_ref, cnt_ref, z_ref, p_ref, wfc_ref, bfc_ref, out_ref,
               psum, pcnt):
    i = pl.program_id(0)

    @pl.when(i == 0)
    def _():
        psum[...] = jnp.zeros_like(psum)
        pcnt[...] = jnp.zeros_like(pcnt)

    d = 1.0 / jnp.maximum(cnt_ref[:, :1], 1.0)
    h = jnp.concatenate([agg_ref[0], agg_ref[1]], axis=1) * d + z_ref[...]
    p = p_ref[...]
    psum[...] += jax.lax.dot_general(p, h, (((1,), (0,)), ((), ())),
                                     preferred_element_type=jnp.float32)
    pcnt[...] += jnp.sum(p, axis=1, keepdims=True)

    @pl.when(i == GRID - 1)
    def _():
        pooled = psum[...] / jnp.maximum(pcnt[:, :1], 1.0)
        out_ref[...] = jax.lax.dot_general(
            pooled, wfc_ref[...], (((1,), (1,)), ((), ())),
            preferred_element_type=jnp.float32) + bfc_ref[...]


def _lin_call(xp, Wl, Wr, bl):
    return pl.pallas_call(
        _lin_body,
        grid=(GRID,),
        in_specs=[
            pl.BlockSpec((BN, F), lambda i: (i, 0)),
            pl.BlockSpec((H, F), lambda i: (0, 0)),
            pl.BlockSpec((H, F), lambda i: (0, 0)),
            pl.BlockSpec((1, H), lambda i: (0, 0)),
        ],
        out_specs=[
            pl.BlockSpec((NC, BN, 128), lambda i: (0, i, 0)),
            pl.BlockSpec((BN, H), lambda i: (i, 0)),
        ],
        out_shape=[
            jax.ShapeDtypeStruct((NC, NP, 128), jnp.float32),
            jax.ShapeDtypeStruct((NP, H), jnp.float32),
        ],
    )(xp, Wl, Wr, bl.reshape(1, H))


def _mid_call(agg, cnt, z, Wl, Wr, bl):
    return pl.pallas_call(
        _mid_body,
        grid=(GRID,),
        in_specs=[
            pl.BlockSpec((NC, BN, 128), lambda i: (0, i, 0)),
            pl.BlockSpec((BN, 128), lambda i: (i, 0)),
            pl.BlockSpec((BN, H), lambda i: (i, 0)),
            pl.BlockSpec((H, H), lambda i: (0, 0)),
            pl.BlockSpec((H, H), lambda i: (0, 0)),
            pl.BlockSpec((1, H), lambda i: (0, 0)),
        ],
        out_specs=[
            pl.BlockSpec((NC, BN, 128), lambda i: (0, i, 0)),
            pl.BlockSpec((BN, H), lambda i: (i, 0)),
        ],
        out_shape=[
            jax.ShapeDtypeStruct((NC, NP, 128), jnp.float32),
            jax.ShapeDtypeStruct((NP, H), jnp.float32),
        ],
    )(agg, cnt, z, Wl, Wr, bl.reshape(1, H))


def _pool_call(agg, cnt, z, P01, Wfc, bfc):
    return pl.pallas_call(
        _pool_body,
        grid=(GRID,),
        in_specs=[
            pl.BlockSpec((NC, BN, 128), lambda i: (0, i, 0)),
            pl.BlockSpec((BN, 128), lambda i: (i, 0)),
            pl.BlockSpec((BN, H), lambda i: (i, 0)),
            pl.BlockSpec((G, BN), lambda i: (0, i)),
            pl.BlockSpec((C, H), lambda i: (0, 0)),
            pl.BlockSpec((1, C), lambda i: (0, 0)),
        ],
        out_specs=pl.BlockSpec((G, C), lambda i: (0, 0)),
        out_shape=jax.ShapeDtypeStruct((G, C), jnp.float32),
        scratch_shapes=[
            pltpu.VMEM((G, H), jnp.float32),
            pltpu.VMEM((G, 128), jnp.float32),
        ],
    )(agg, cnt, z, P01, Wfc, bfc.reshape(1, C))


def kernel(x, edge_index, batch, Wl1, bl1, Wr1, Wl2, bl2, Wr2, Wl3, bl3, Wr3,
           Wfc, bfc):
    xp = jnp.pad(x, ((0, NP - N), (0, 0)))
    src_flat = jnp.pad(edge_index[0], (0, EP - E))
    src = jnp.stack([src_flat, src_flat + NP]).reshape(NC, NS, NCHUNK, CH)
    dst = jnp.pad(edge_index[1], (0, EP - E), constant_values=N).reshape(
        NS, NCHUNK, CH)
    batch_p = jnp.pad(batch, (0, NP - N), constant_values=G)
    P01 = (batch_p[None, :] == jnp.arange(G, dtype=batch.dtype)[:, None]
           ).astype(jnp.float32)
    ones128 = jnp.ones((CH, 128), jnp.float32)
    zeros128 = jnp.zeros((ROWS_PER_TILE, 128), jnp.float32)

    cnt = _count_kernel(dst, ones128, zeros128)

    ya, z = _lin_call(xp, Wl1, Wr1, bl1)
    agg = _segsum_kernel(ya.reshape(NC * NP, 128), src, dst, zeros128)
    ya, z = _mid_call(agg, cnt, z, Wl2, Wr2, bl2)
    agg = _segsum_kernel(ya.reshape(NC * NP, 128), src, dst, zeros128)
    ya, z = _mid_call(agg, cnt, z, Wl3, Wr3, bl3)
    agg = _segsum_kernel(ya.reshape(NC * NP, 128), src, dst, zeros128)
    return _pool_call(agg, cnt, z, P01, Wfc, bfc)


# speedup vs baseline: 1.1539x; 1.1117x over previous
import jax
import jax.numpy as jnp
from jax import lax
from jax.experimental import pallas as pl
from jax.experimental.pallas import tpu as pltpu
from jax.experimental.pallas import tpu_sc as plsc

N = 10000
E = 160000
F = 256
H = 256
C = 40
G = 16

NC = 2
NS = 16
CH = 64

NP = 10240
ROWS_PER_TILE = NP // NS
NCHUNK = 160
EP = NS * NCHUNK * CH
BN = 1024
GRID = NP // BN

_mesh = plsc.VectorSubcoreMesh(
    core_axis_name="c", subcore_axis_name="s", num_cores=NC, num_subcores=NS
)


def _count_body(dst_hbm, ones_hbm, zeros_hbm, cnt_hbm, dst_v, ones_v, shared):
    c = lax.axis_index("c")
    s = lax.axis_index("s")
    r0 = s * ROWS_PER_TILE

    pltpu.sync_copy(zeros_hbm, shared.at[pl.ds(r0, ROWS_PER_TILE)])
    pltpu.sync_copy(dst_hbm.at[s, pl.ds(c * (NCHUNK // NC), NCHUNK // NC)],
                    dst_v)
    pltpu.sync_copy(ones_hbm, ones_v)
    plsc.subcore_barrier()

    def body(j, carry):
        pltpu.sync_copy(ones_v, shared.at[dst_v.at[j]], add=True)
        return carry
    lax.fori_loop(0, NCHUNK // NC, body, 0)

    plsc.subcore_barrier()
    pltpu.sync_copy(shared.at[pl.ds(r0, ROWS_PER_TILE)],
                    cnt_hbm.at[c, pl.ds(r0, ROWS_PER_TILE)])


_count_kernel = pl.kernel(
    _count_body,
    mesh=_mesh,
    out_type=jax.ShapeDtypeStruct((NC, NP, 128), jnp.float32),
    scratch_types=[
        pltpu.VMEM((NCHUNK // NC, CH), jnp.int32),
        pltpu.VMEM((CH, 128), jnp.float32),
        pltpu.VMEM_SHARED((NP, 128), jnp.float32),
    ],
)


NBUF = 4
NPH = 4
CPP = NCHUNK // NPH


def _segsum_body(ya_hbm, src_hbm, dst_hbm, zeros_hbm, agg_hbm,
                 src_v, dst_v, buf, gsems, ssems, shared):
    c = lax.axis_index("c")
    s = lax.axis_index("s")
    r0 = s * ROWS_PER_TILE

    pltpu.sync_copy(zeros_hbm, shared.at[pl.ds(r0, ROWS_PER_TILE)])
    plsc.subcore_barrier()

    def gather_start(j, b):
        pltpu.async_copy(ya_hbm.at[src_v.at[j]], buf.at[b], gsems.at[b])

    def gather_wait(j, b):
        pltpu.make_async_copy(ya_hbm.at[src_v.at[j]], buf.at[b],
                              gsems.at[b]).wait()

    def scatter_start(j, b):
        pltpu.async_copy(buf.at[b], shared.at[dst_v.at[j]], ssems.at[b],
                         add=True)

    def scatter_wait(j, b):
        pltpu.make_async_copy(buf.at[b], shared.at[dst_v.at[j]],
                              ssems.at[b]).wait()

    for ph in range(NPH):
        pltpu.sync_copy(src_hbm.at[c, s, pl.ds(ph * CPP, CPP)], src_v)
        pltpu.sync_copy(dst_hbm.at[s, pl.ds(ph * CPP, CPP)], dst_v)

        gather_start(0, 0)
        gather_start(1, 1)

        def body(jj, carry):
            j0 = NBUF * jj
            for b in range(NBUF):
                j = j0 + b
                gather_wait(j, b)
                scatter_start(j, b)

                @pl.when(j >= 2)
                def _():
                    scatter_wait(j - 2, (b + 2) % NBUF)

                @pl.when(j + 2 < CPP)
                def _():
                    gather_start(j + 2, (b + 2) % NBUF)

            return carry

        lax.fori_loop(0, CPP // NBUF, body, 0)
        scatter_wait(CPP - 2, (CPP - 2) % NBUF)
        scatter_wait(CPP - 1, (CPP - 1) % NBUF)

    plsc.subcore_barrier()

    pltpu.sync_copy(shared.at[pl.ds(r0, ROWS_PER_TILE)],
                    agg_hbm.at[c, pl.ds(r0, ROWS_PER_TILE)])


_segsum_kernel = pl.kernel(
    _segsum_body,
    mesh=_mesh,
    out_type=jax.ShapeDtypeStruct((NC, NP, 128), jnp.float32),
    scratch_types=[
        pltpu.VMEM((CPP, CH), jnp.int32),
        pltpu.VMEM((CPP, CH), jnp.int32),
        pltpu.VMEM((NBUF, CH, 128), jnp.float32),
        pltpu.SemaphoreType.DMA((NBUF,)),
        pltpu.SemaphoreType.DMA((NBUF,)),
        pltpu.VMEM_SHARED((NP, 128), jnp.float32),
    ],
)


def _liny_body(x_ref, wl_ref, ya_ref):
    y = jax.lax.dot_general(x_ref[...], wl_ref[...], (((1,), (1,)), ((), ())),
                            preferred_element_type=jnp.float32)
    ya_ref[0] = y[:, :128]
    ya_ref[1] = y[:, 128:]


def _linz_body(x_ref, wr_ref, bl_ref, z_ref):
    z_ref[...] = jax.lax.dot_general(x_ref[...], wr_ref[...],
                                     (((1,), (1,)), ((), ())),
                                     preferred_element_type=jnp.float32) + bl_ref[...]


def _midy_body(agg_ref, cnt_ref, z_ref, wl_ref, h_ref, ya_ref):
    d = 1.0 / jnp.maximum(cnt_ref[0, :, :1] + cnt_ref[1, :, :1], 1.0)
    h = jnp.concatenate([agg_ref[0], agg_ref[1]], axis=1) * d + z_ref[...]
    h = jnp.maximum(h, 0.0)
    h_ref[...] = h
    y = jax.lax.dot_general(h, wl_ref[...], (((1,), (1,)), ((), ())),
                            preferred_element_type=jnp.float32)
    ya_ref[0] = y[:, :128]
    ya_ref[1] = y[:, 128:]


def _pool_body(agg_ref, cnt_ref, z_ref, p_ref, wfc_ref, bfc_ref, out_ref,
               psum, pcnt):
    i = pl.program_id(0)

    @pl.when(i == 0)
    def _():
        psum[...] = jnp.zeros_like(psum)
        pcnt[...] = jnp.zeros_like(pcnt)

    d = 1.0 / jnp.maximum(cnt_ref[0, :, :1] + cnt_ref[1, :, :1], 1.0)
    h = jnp.concatenate([agg_ref[0], agg_ref[1]], axis=1) * d + z_ref[...]
    p = p_ref[...]
    psum[...] += jax.lax.dot_general(p, h, (((1,), (0,)), ((), ())),
                                     preferred_element_type=jnp.float32)
    pcnt[...] += jnp.sum(p, axis=1, keepdims=True)

    @pl.when(i == GRID - 1)
    def _():
        pooled = psum[...] / jnp.maximum(pcnt[:, :1], 1.0)
        out_ref[...] = jax.lax.dot_general(
            pooled, wfc_ref[...], (((1,), (1,)), ((), ())),
            preferred_element_type=jnp.float32) + bfc_ref[...]


def _liny_call(xp, Wl):
    return pl.pallas_call(
        _liny_body,
        grid=(GRID,),
        in_specs=[
            pl.BlockSpec((BN, F), lambda i: (i, 0)),
            pl.BlockSpec((H, F), lambda i: (0, 0)),
        ],
        out_specs=pl.BlockSpec((NC, BN, 128), lambda i: (0, i, 0)),
        out_shape=jax.ShapeDtypeStruct((NC, NP, 128), jnp.float32),
    )(xp, Wl)


def _linz_call(xp, Wr, bl):
    return pl.pallas_call(
        _linz_body,
        grid=(GRID,),
        in_specs=[
            pl.BlockSpec((BN, F), lambda i: (i, 0)),
            pl.BlockSpec((H, F), lambda i: (0, 0)),
            pl.BlockSpec((1, H), lambda i: (0, 0)),
        ],
        out_specs=pl.BlockSpec((BN, H), lambda i: (i, 0)),
        out_shape=jax.ShapeDtypeStruct((NP, H), jnp.float32),
    )(xp, Wr, bl.reshape(1, H))


def _midy_call(agg, cnt, z, Wl):
    return pl.pallas_call(
        _midy_body,
        grid=(GRID,),
        in_specs=[
            pl.BlockSpec((NC, BN, 128), lambda i: (0, i, 0)),
            pl.BlockSpec((NC, BN, 128), lambda i: (0, i, 0)),
            pl.BlockSpec((BN, H), lambda i: (i, 0)),
            pl.BlockSpec((H, H), lambda i: (0, 0)),
        ],
        out_specs=[
            pl.BlockSpec((BN, H), lambda i: (i, 0)),
            pl.BlockSpec((NC, BN, 128), lambda i: (0, i, 0)),
        ],
        out_shape=[
            jax.ShapeDtypeStruct((NP, H), jnp.float32),
            jax.ShapeDtypeStruct((NC, NP, 128), jnp.float32),
        ],
    )(agg, cnt, z, Wl)


def _pool_call(agg, cnt, z, P01, Wfc, bfc):
    return pl.pallas_call(
        _pool_body,
        grid=(GRID,),
        in_specs=[
            pl.BlockSpec((NC, BN, 128), lambda i: (0, i, 0)),
            pl.BlockSpec((NC, BN, 128), lambda i: (0, i, 0)),
            pl.BlockSpec((BN, H), lambda i: (i, 0)),
            pl.BlockSpec((G, BN), lambda i: (0, i)),
            pl.BlockSpec((C, H), lambda i: (0, 0)),
            pl.BlockSpec((1, C), lambda i: (0, 0)),
        ],
        out_specs=pl.BlockSpec((G, C), lambda i: (0, 0)),
        out_shape=jax.ShapeDtypeStruct((G, C), jnp.float32),
        scratch_shapes=[
            pltpu.VMEM((G, H), jnp.float32),
            pltpu.VMEM((G, 128), jnp.float32),
        ],
    )(agg, cnt, z, P01, Wfc, bfc.reshape(1, C))


def kernel(x, edge_index, batch, Wl1, bl1, Wr1, Wl2, bl2, Wr2, Wl3, bl3, Wr3,
           Wfc, bfc):
    xp = jnp.pad(x, ((0, NP - N), (0, 0)))
    src_flat = jnp.pad(edge_index[0], (0, EP - E))
    src = jnp.stack([src_flat, src_flat + NP]).reshape(NC, NS, NCHUNK, CH)
    dst = jnp.pad(edge_index[1], (0, EP - E), constant_values=N).reshape(
        NS, NCHUNK, CH)
    batch_p = jnp.pad(batch, (0, NP - N), constant_values=G)
    P01 = (batch_p[None, :] == jnp.arange(G, dtype=batch.dtype)[:, None]
           ).astype(jnp.float32)
    ones128 = jnp.ones((CH, 128), jnp.float32)
    zeros128 = jnp.zeros((ROWS_PER_TILE, 128), jnp.float32)

    cnt = _count_kernel(dst, ones128, zeros128)

    ya = _liny_call(xp, Wl1)
    z = _linz_call(xp, Wr1, bl1)
    agg = _segsum_kernel(ya.reshape(NC * NP, 128), src, dst, zeros128)

    h, ya = _midy_call(agg, cnt, z, Wl2)
    z = _linz_call(h, Wr2, bl2)
    agg = _segsum_kernel(ya.reshape(NC * NP, 128), src, dst, zeros128)

    h, ya = _midy_call(agg, cnt, z, Wl3)
    z = _linz_call(h, Wr3, bl3)
    agg = _segsum_kernel(ya.reshape(NC * NP, 128), src, dst, zeros128)

    return _pool_call(agg, cnt, z, P01, Wfc, bfc)
